# baseline (speedup 1.0000x reference)
import functools

import jax
import jax.numpy as jnp
from jax import lax
from jax.experimental import pallas as pl
from jax.experimental.pallas import tpu as pltpu
from jax.experimental.pallas import tpu_sc as plsc

N = 10000
E = 320000
D = 128
H = 128
L = 128
ITERS = 3

NC = 2
NS = 16
CH = 64
NW = NC * NS
CH_PER_W = 4 * (-(-E // (4 * CH * NW)))
EPAD = CH_PER_W * CH * NW
NCHT = EPAD // CH
CPW0 = 92
CPW1 = 2 * CH_PER_W - CPW0
NPAD = 10112

ROWS_PER_TILE = NPAD // NS
ZSIZES = [CH] * (ROWS_PER_TILE // CH) + (
    [ROWS_PER_TILE % CH] if ROWS_PER_TILE % CH else [])


def _sc_edge_body(a_hbm, b_hbm, idx_hbm, out_hbm,
                  idxbuf, arows, brows, mrows, aggsh,
                  sga0, sga1, sgb0, sgb1, ssc0, ssc1,
                  six0, six1, six2, six3):
    cid = lax.axis_index("c")
    sid = lax.axis_index("s")
    ncw = jnp.where(cid == 0, CPW0, CPW1)
    cbase = jnp.where(cid == 0, sid * CPW0, NS * CPW0 + sid * CPW1)
    sga = (sga0, sga1)
    sgb = (sgb0, sgb1)
    ssc = (ssc0, ssc1)
    six = (six0, six1, six2, six3)

    zero = jnp.zeros((16,), jnp.float32)

    def zrow(e, carry):
        for j in range(H // 16):
            mrows[0, e, pl.ds(j * 16, 16)] = zero
        return carry

    lax.fori_loop(0, CH, zrow, 0)
    r0 = sid * ROWS_PER_TILE
    off = 0
    for zn in ZSIZES:
        pltpu.sync_copy(mrows.at[0, pl.ds(0, zn)],
                        aggsh.at[pl.ds(r0 + off, zn)])
        off += zn
    plsc.subcore_barrier()

    def idx_start(ib, c):
        pltpu.async_copy(idx_hbm.at[cbase + c], idxbuf.at[ib], six[ib])

    def idx_wait(ib):
        pltpu.make_async_copy(idx_hbm.at[0], idxbuf.at[ib],
                              six[ib]).wait()

    def gather_start(b, ib):
        pltpu.async_copy(a_hbm.at[idxbuf.at[ib, 0]], arows.at[b], sga[b])
        pltpu.async_copy(b_hbm.at[idxbuf.at[ib, 1]], brows.at[b], sgb[b])

    def gather_wait(b):
        pltpu.make_async_copy(a_hbm.at[pl.ds(0, CH)], arows.at[b],
                              sga[b]).wait()
        pltpu.make_async_copy(b_hbm.at[pl.ds(0, CH)], brows.at[b],
                              sgb[b]).wait()

    def compute(b):
        @plsc.parallel_loop(0, CH, 1, unroll=2)
        def edge(e):
            for j in range(H // 16):
                s = pl.ds(j * 16, 16)
                mrows[b, e, s] = jnp.maximum(arows[b, e, s] + brows[b, e, s],
                                             0.0)

    def scatter_start(b, ib):
        pltpu.async_copy(mrows.at[b], aggsh.at[idxbuf.at[ib, 1]], ssc[b],
                         add=True)

    def scatter_wait(b):
        pltpu.make_async_copy(a_hbm.at[pl.ds(0, CH)], mrows.at[b],
                              ssc[b]).wait()

    def step(c, k, first):
        b = k % 2
        ib = k % 4
        ibn = (k + 2) % 4
        gather_wait(b)
        if not first:
            scatter_wait(b)
            cn2 = jnp.where(c + 2 >= ncw, 0, c + 2)
            idx_start(ibn, cn2)
        compute(b)
        scatter_start(b, ib)
        idx_wait(ibn)
        gather_start(b, ibn)

    for ib in range(4):
        idx_start(ib, jnp.int32(ib))
    idx_wait(0)
    gather_start(0, 0)
    idx_wait(1)
    gather_start(1, 1)
    step(jnp.int32(0), 0, True)
    step(jnp.int32(1), 1, True)
    step(jnp.int32(2), 2, False)
    step(jnp.int32(3), 3, False)

    def group(g, carry):
        for k in range(4):
            step(4 * g + k, k, False)
        return carry

    lax.fori_loop(1, ncw // 4, group, 0)
    for b in (0, 1):
        gather_wait(b)
        scatter_wait(b)
    plsc.subcore_barrier()

    off = 0
    for zn in ZSIZES:
        pltpu.sync_copy(aggsh.at[pl.ds(r0 + off, zn)],
                        out_hbm.at[cid, pl.ds(r0 + off, zn)])
        off += zn


@functools.lru_cache(maxsize=1)
def _sc_edge():
    return functools.partial(
        pl.kernel,
        mesh=plsc.VectorSubcoreMesh(core_axis_name="c", subcore_axis_name="s"),
        out_type=jax.ShapeDtypeStruct((NC, NPAD, H), jnp.float32),
        scratch_types=[
            pltpu.VMEM((4, 2, CH), jnp.int32),

            pltpu.VMEM((2, CH, H), jnp.float32),
            pltpu.VMEM((2, CH, H), jnp.float32),
            pltpu.VMEM((2, CH, H), jnp.float32),
            pltpu.VMEM_SHARED((NPAD, H), jnp.float32),
        ] + [pltpu.SemaphoreType.DMA] * 10,
    )(_sc_edge_body)


BLK = 1000
GRID = N // BLK


def _mm(a, b):
    return jnp.dot(a, b, preferred_element_type=jnp.float32)


def _enc_kernel(x_ref, w1, b1, w2, b2, wmt, wmb, bm, h_ref, a_ref, b_ref):
    h1 = jnp.maximum(_mm(x_ref[...], w1[...]) + b1[...], 0.0)
    h = jnp.maximum(_mm(h1, w2[...]) + b2[...], 0.0)
    h_ref[...] = h
    a_ref[...] = _mm(h, wmt[...]) + bm[...]
    b_ref[...] = _mm(h, wmb[...])


def _upd_kernel(h_ref, agg_ref, wut, wub, bu, wmt, wmb, bm,
                hn_ref, a_ref, b_ref):
    agg = agg_ref[0] + agg_ref[1]
    hn = jnp.maximum(_mm(h_ref[...], wut[...]) + _mm(agg, wub[...]) + bu[...],
                     0.0)
    hn_ref[...] = hn
    a_ref[...] = _mm(hn, wmt[...]) + bm[...]
    b_ref[...] = _mm(hn, wmb[...])


def _upd_last_kernel(h_ref, agg_ref, wut, wub, bu, wr1, br1, wr2, br2,
                     out_ref, gacc):
    i = pl.program_id(0)
    agg = agg_ref[0] + agg_ref[1]
    hn = jnp.maximum(_mm(h_ref[...], wut[...]) + _mm(agg, wub[...]) + bu[...],
                     0.0)
    part = jnp.sum(hn.reshape(BLK // 8, 8, H), axis=0)

    @pl.when(i == 0)
    def _init():
        gacc[...] = part

    @pl.when(i > 0)
    def _acc():
        gacc[...] = gacc[...] + part

    @pl.when(i == GRID - 1)
    def _readout():
        g = jnp.sum(gacc[...], axis=0, keepdims=True)
        lat = _mm(jnp.maximum(_mm(g, wr1[...]) + br1[...], 0.0), wr2[...])
        out_ref[...] = lat + br2[...]


def _row_spec():
    return pl.BlockSpec((BLK, H), lambda i: (i, 0))


def _full_spec(shape):
    return pl.BlockSpec(shape, lambda i: tuple(0 for _ in shape))


_NODE_SHAPE = jax.ShapeDtypeStruct((N, H), jnp.float32)

_enc_call = pl.pallas_call(
    _enc_kernel,
    grid=(GRID,),
    in_specs=[_row_spec()] + [_full_spec(s) for s in
                              [(D, H), (1, H), (H, H), (1, H),
                               (H, H), (H, H), (1, H)]],
    out_specs=[_row_spec(), _row_spec(), _row_spec()],
    out_shape=[_NODE_SHAPE, _NODE_SHAPE, _NODE_SHAPE],
)

_upd_call = pl.pallas_call(
    _upd_kernel,
    grid=(GRID,),
    in_specs=[_row_spec(),
              pl.BlockSpec((NC, BLK, H), lambda i: (0, i, 0))] +
             [_full_spec(s) for s in
              [(H, H), (H, H), (1, H), (H, H), (H, H), (1, H)]],
    out_specs=[_row_spec(), _row_spec(), _row_spec()],
    out_shape=[_NODE_SHAPE, _NODE_SHAPE, _NODE_SHAPE],
)

_upd_last_call = pl.pallas_call(
    _upd_last_kernel,
    grid=(GRID,),
    in_specs=[_row_spec(),
              pl.BlockSpec((NC, BLK, H), lambda i: (0, i, 0))] +
             [_full_spec(s) for s in
              [(H, H), (H, H), (1, H), (H, H), (1, H), (H, L), (1, L)]],
    out_specs=pl.BlockSpec((1, L), lambda i: (0, 0)),
    out_shape=jax.ShapeDtypeStruct((1, L), jnp.float32),
    scratch_shapes=[pltpu.VMEM((8, H), jnp.float32)],
)


def kernel(x, edge_index, W_enc1, b_enc1, W_enc2, b_enc2, W_msg, b_msg,
           W_upd, b_upd, W_r1, b_r1, W_r2, b_r2):
    src = edge_index[0].astype(jnp.int32)
    dst = edge_index[1].astype(jnp.int32)
    npd = EPAD - E
    dst_pad = N + jnp.arange(npd, dtype=jnp.int32) % (NPAD - N)
    src_p = jnp.concatenate([src, jnp.zeros((npd,), jnp.int32)])
    dst_p = jnp.concatenate([dst, dst_pad])
    idx_p = jnp.stack([src_p.reshape(NCHT, CH),
                       dst_p.reshape(NCHT, CH)], axis=1)

    wmt, wmb = W_msg[:H], W_msg[H:]
    wut, wub = W_upd[:H], W_upd[H:]
    b1 = b_enc1.reshape(1, H)
    b2 = b_enc2.reshape(1, H)
    bm = b_msg.reshape(1, H)
    bu = b_upd.reshape(1, H)
    br1 = b_r1.reshape(1, H)
    br2 = b_r2.reshape(1, L)

    h, a, b = _enc_call(x, W_enc1, b1, W_enc2, b2, wmt, wmb, bm)
    for it in range(ITERS):
        aggp = _sc_edge()(a, b, idx_p)
        if it < ITERS - 1:
            h, a, b = _upd_call(h, aggp, wut, wub, bu, wmt, wmb, bm)
        else:
            lat = _upd_last_call(h, aggp, wut, wub, bu,
                                 W_r1, br1, W_r2, br2)
    return lat.reshape(L)


# speedup vs baseline: 1.1454x; 1.1454x over previous
import functools

import jax
import jax.numpy as jnp
from jax import lax
from jax.experimental import pallas as pl
from jax.experimental.pallas import tpu as pltpu
from jax.experimental.pallas import tpu_sc as plsc

N = 10000
E = 320000
D = 128
H = 128
L = 128
ITERS = 3

NC = 2
NS = 16
CH = 64
NW = NC * NS
CH_PER_W = 4 * (-(-E // (4 * CH * NW)))
EPAD = CH_PER_W * CH * NW
NCHT = EPAD // CH
CPW0 = 228
CPW1 = 2 * CH_PER_W - CPW0
NPAD = 10112

ROWS_PER_TILE = NPAD // NS
ZSIZES = [CH] * (ROWS_PER_TILE // CH) + (
    [ROWS_PER_TILE % CH] if ROWS_PER_TILE % CH else [])


def _sc_edge_body(a_hbm, b_hbm, idx_hbm, out_hbm,
                  idxbuf, arows, brows, mrows, aggsh,
                  sga0, sga1, sgb0, sgb1, ssc0, ssc1,
                  six0, six1, six2, six3):
    cid = lax.axis_index("c")
    sid = lax.axis_index("s")
    ncw = jnp.where(cid == 0, CPW0, CPW1)
    cbase = jnp.where(cid == 0, sid * CPW0, NS * CPW0 + sid * CPW1)
    sga = (sga0, sga1)
    sgb = (sgb0, sgb1)
    ssc = (ssc0, ssc1)
    six = (six0, six1, six2, six3)

    zero = jnp.zeros((16,), jnp.float32)

    def zrow(e, carry):
        for j in range(H // 16):
            mrows[0, e, pl.ds(j * 16, 16)] = zero
        return carry

    lax.fori_loop(0, CH, zrow, 0)
    r0 = sid * ROWS_PER_TILE
    off = 0
    for zn in ZSIZES:
        pltpu.sync_copy(mrows.at[0, pl.ds(0, zn)],
                        aggsh.at[pl.ds(r0 + off, zn)])
        off += zn
    plsc.subcore_barrier()

    def idx_start(ib, c):
        pltpu.async_copy(idx_hbm.at[cbase + c], idxbuf.at[ib], six[ib])

    def idx_wait(ib):
        pltpu.make_async_copy(idx_hbm.at[0], idxbuf.at[ib],
                              six[ib]).wait()

    def gather_start(b, ib):
        pltpu.async_copy(a_hbm.at[idxbuf.at[ib, 0]], arows.at[b], sga[b])
        pltpu.async_copy(b_hbm.at[idxbuf.at[ib, 1]], brows.at[b], sgb[b])

    def gather_wait(b):
        pltpu.make_async_copy(a_hbm.at[pl.ds(0, CH)], arows.at[b],
                              sga[b]).wait()
        pltpu.make_async_copy(b_hbm.at[pl.ds(0, CH)], brows.at[b],
                              sgb[b]).wait()

    def compute(b):
        @plsc.parallel_loop(0, CH, 1, unroll=2)
        def edge(e):
            for j in range(H // 16):
                s = pl.ds(j * 16, 16)
                mrows[b, e, s] = jnp.maximum(arows[b, e, s] + brows[b, e, s],
                                             0.0)

    def scatter_start(b, ib):
        pltpu.async_copy(mrows.at[b], aggsh.at[idxbuf.at[ib, 1]], ssc[b],
                         add=True)

    def scatter_wait(b):
        pltpu.make_async_copy(a_hbm.at[pl.ds(0, CH)], mrows.at[b],
                              ssc[b]).wait()

    def step(c, k, first):
        b = k % 2
        ib = k % 4
        ibn = (k + 2) % 4
        gather_wait(b)
        if not first:
            scatter_wait(b)
            cn2 = jnp.where(c + 2 >= ncw, 0, c + 2)
            idx_start(ibn, cn2)
        compute(b)
        scatter_start(b, ib)
        idx_wait(ibn)
        gather_start(b, ibn)

    for ib in range(4):
        idx_start(ib, jnp.int32(ib))
    idx_wait(0)
    gather_start(0, 0)
    idx_wait(1)
    gather_start(1, 1)
    step(jnp.int32(0), 0, True)
    step(jnp.int32(1), 1, True)
    step(jnp.int32(2), 2, False)
    step(jnp.int32(3), 3, False)

    def group(g, carry):
        for k in range(4):
            step(4 * g + k, k, False)
        return carry

    lax.fori_loop(1, ncw // 4, group, 0)
    for b in (0, 1):
        gather_wait(b)
        scatter_wait(b)
    plsc.subcore_barrier()

    off = 0
    for zn in ZSIZES:
        pltpu.sync_copy(aggsh.at[pl.ds(r0 + off, zn)],
                        out_hbm.at[cid, pl.ds(r0 + off, zn)])
        off += zn


@functools.lru_cache(maxsize=1)
def _sc_edge():
    return functools.partial(
        pl.kernel,
        mesh=plsc.VectorSubcoreMesh(core_axis_name="c", subcore_axis_name="s"),
        out_type=jax.ShapeDtypeStruct((NC, NPAD, H), jnp.float32),
        scratch_types=[
            pltpu.VMEM((4, 2, CH), jnp.int32),

            pltpu.VMEM((2, CH, H), jnp.float32),
            pltpu.VMEM((2, CH, H), jnp.float32),
            pltpu.VMEM((2, CH, H), jnp.float32),
            pltpu.VMEM_SHARED((NPAD, H), jnp.float32),
        ] + [pltpu.SemaphoreType.DMA] * 10,
    )(_sc_edge_body)


BLK = 1000
GRID = N // BLK


def _mm(a, b):
    return jnp.dot(a, b, preferred_element_type=jnp.float32)


def _enc_kernel(x_ref, w1, b1, w2, b2, wmt, wmb, bm, h_ref, a_ref, b_ref):
    h1 = jnp.maximum(_mm(x_ref[...], w1[...]) + b1[...], 0.0)
    h = jnp.maximum(_mm(h1, w2[...]) + b2[...], 0.0)
    h_ref[...] = h
    a_ref[...] = _mm(h, wmt[...]) + bm[...]
    b_ref[...] = _mm(h, wmb[...])


def _upd_kernel(h_ref, agg_ref, wut, wub, bu, wmt, wmb, bm,
                hn_ref, a_ref, b_ref):
    agg = agg_ref[0] + agg_ref[1]
    hn = jnp.maximum(_mm(h_ref[...], wut[...]) + _mm(agg, wub[...]) + bu[...],
                     0.0)
    hn_ref[...] = hn
    a_ref[...] = _mm(hn, wmt[...]) + bm[...]
    b_ref[...] = _mm(hn, wmb[...])


def _upd_last_kernel(h_ref, agg_ref, wut, wub, bu, wr1, br1, wr2, br2,
                     out_ref, gacc):
    i = pl.program_id(0)
    agg = agg_ref[0] + agg_ref[1]
    hn = jnp.maximum(_mm(h_ref[...], wut[...]) + _mm(agg, wub[...]) + bu[...],
                     0.0)
    part = jnp.sum(hn.reshape(BLK // 8, 8, H), axis=0)

    @pl.when(i == 0)
    def _init():
        gacc[...] = part

    @pl.when(i > 0)
    def _acc():
        gacc[...] = gacc[...] + part

    @pl.when(i == GRID - 1)
    def _readout():
        g = jnp.sum(gacc[...], axis=0, keepdims=True)
        lat = _mm(jnp.maximum(_mm(g, wr1[...]) + br1[...], 0.0), wr2[...])
        out_ref[...] = lat + br2[...]


def _row_spec():
    return pl.BlockSpec((BLK, H), lambda i: (i, 0))


def _full_spec(shape):
    return pl.BlockSpec(shape, lambda i: tuple(0 for _ in shape))


_NODE_SHAPE = jax.ShapeDtypeStruct((N, H), jnp.float32)

_enc_call = pl.pallas_call(
    _enc_kernel,
    grid=(GRID,),
    in_specs=[_row_spec()] + [_full_spec(s) for s in
                              [(D, H), (1, H), (H, H), (1, H),
                               (H, H), (H, H), (1, H)]],
    out_specs=[_row_spec(), _row_spec(), _row_spec()],
    out_shape=[_NODE_SHAPE, _NODE_SHAPE, _NODE_SHAPE],
)

_upd_call = pl.pallas_call(
    _upd_kernel,
    grid=(GRID,),
    in_specs=[_row_spec(),
              pl.BlockSpec((NC, BLK, H), lambda i: (0, i, 0))] +
             [_full_spec(s) for s in
              [(H, H), (H, H), (1, H), (H, H), (H, H), (1, H)]],
    out_specs=[_row_spec(), _row_spec(), _row_spec()],
    out_shape=[_NODE_SHAPE, _NODE_SHAPE, _NODE_SHAPE],
)

_upd_last_call = pl.pallas_call(
    _upd_last_kernel,
    grid=(GRID,),
    in_specs=[_row_spec(),
              pl.BlockSpec((NC, BLK, H), lambda i: (0, i, 0))] +
             [_full_spec(s) for s in
              [(H, H), (H, H), (1, H), (H, H), (1, H), (H, L), (1, L)]],
    out_specs=pl.BlockSpec((1, L), lambda i: (0, 0)),
    out_shape=jax.ShapeDtypeStruct((1, L), jnp.float32),
    scratch_shapes=[pltpu.VMEM((8, H), jnp.float32)],
)


def kernel(x, edge_index, W_enc1, b_enc1, W_enc2, b_enc2, W_msg, b_msg,
           W_upd, b_upd, W_r1, b_r1, W_r2, b_r2):
    src = edge_index[0].astype(jnp.int32)
    dst = edge_index[1].astype(jnp.int32)
    npd = EPAD - E
    dst_pad = N + jnp.arange(npd, dtype=jnp.int32) % (NPAD - N)
    src_p = jnp.concatenate([src, jnp.zeros((npd,), jnp.int32)])
    dst_p = jnp.concatenate([dst, dst_pad])
    idx_p = jnp.stack([src_p.reshape(NCHT, CH),
                       dst_p.reshape(NCHT, CH)], axis=1)

    wmt, wmb = W_msg[:H], W_msg[H:]
    wut, wub = W_upd[:H], W_upd[H:]
    b1 = b_enc1.reshape(1, H)
    b2 = b_enc2.reshape(1, H)
    bm = b_msg.reshape(1, H)
    bu = b_upd.reshape(1, H)
    br1 = b_r1.reshape(1, H)
    br2 = b_r2.reshape(1, L)

    h, a, b = _enc_call(x, W_enc1, b1, W_enc2, b2, wmt, wmb, bm)
    for it in range(ITERS):
        aggp = _sc_edge()(a, b, idx_p)
        if it < ITERS - 1:
            h, a, b = _upd_call(h, aggp, wut, wub, bu, wmt, wmb, bm)
        else:
            lat = _upd_last_call(h, aggp, wut, wub, bu,
                                 W_r1, br1, W_r2, br2)
    return lat.reshape(L)


# speedup vs baseline: 1.8225x; 1.5911x over previous
import functools

import jax
import jax.numpy as jnp
import numpy as np
from jax import lax
from jax.experimental import pallas as pl
from jax.experimental.pallas import tpu as pltpu
from jax.experimental.pallas import tpu_sc as plsc

N = 10000
E = 320000
D = 128
H = 128
L = 128
ITERS = 3

NC = 2
NS = 16
CH = 64
NW = NC * NS
CH_PER_W = 4 * (-(-E // (4 * CH * NW)))
EPAD = CH_PER_W * CH * NW
NCHT = EPAD // CH
CPW0 = 228
CPW1 = 2 * CH_PER_W - CPW0
NPAD = 10112

ROWS_PER_TILE = NPAD // NS
PERM = np.arange(H).reshape(H // 32, 16, 2).transpose(0, 2, 1).reshape(H)
ZSIZES = [CH] * (ROWS_PER_TILE // CH) + (
    [ROWS_PER_TILE % CH] if ROWS_PER_TILE % CH else [])


def _sc_edge_body(a_hbm, b_hbm, idx_hbm, out_hbm,
                  idxbuf, arows, brows, mrows, aggsh,
                  sga0, sga1, sgb0, sgb1, ssc0, ssc1,
                  six0, six1, six2, six3):
    cid = lax.axis_index("c")
    sid = lax.axis_index("s")
    ncw = jnp.where(cid == 0, CPW0, CPW1)
    cbase = jnp.where(cid == 0, sid * CPW0, NS * CPW0 + sid * CPW1)
    sga = (sga0, sga1)
    sgb = (sgb0, sgb1)
    ssc = (ssc0, ssc1)
    six = (six0, six1, six2, six3)

    zero = jnp.zeros((16,), jnp.float32)

    def zrow(e, carry):
        for j in range(H // 16):
            mrows[0, e, pl.ds(j * 16, 16)] = zero
        return carry

    lax.fori_loop(0, CH, zrow, 0)
    r0 = sid * ROWS_PER_TILE
    off = 0
    for zn in ZSIZES:
        pltpu.sync_copy(mrows.at[0, pl.ds(0, zn)],
                        aggsh.at[pl.ds(r0 + off, zn)])
        off += zn
    plsc.subcore_barrier()

    def idx_start(ib, c):
        pltpu.async_copy(idx_hbm.at[cbase + c], idxbuf.at[ib], six[ib])

    def idx_wait(ib):
        pltpu.make_async_copy(idx_hbm.at[0], idxbuf.at[ib],
                              six[ib]).wait()

    def gather_start(b, ib):
        pltpu.async_copy(a_hbm.at[idxbuf.at[ib, 0]], arows.at[b], sga[b])
        pltpu.async_copy(b_hbm.at[idxbuf.at[ib, 1]], brows.at[b], sgb[b])

    def gather_wait(b):
        pltpu.make_async_copy(a_hbm.at[pl.ds(0, CH)], arows.at[b],
                              sga[b]).wait()
        pltpu.make_async_copy(b_hbm.at[pl.ds(0, CH)], brows.at[b],
                              sgb[b]).wait()

    def compute(b):
        msk = jnp.full((16,), -65536, jnp.int32)

        @plsc.parallel_loop(0, CH, 1, unroll=2)
        def edge(e):
            for j in range(H // 32):
                sw = pl.ds(j * 16, 16)
                va = arows[b, e, sw]
                vb = brows[b, e, sw]
                fae = lax.bitcast_convert_type(va << 16, jnp.float32)
                fao = lax.bitcast_convert_type(va & msk, jnp.float32)
                fbe = lax.bitcast_convert_type(vb << 16, jnp.float32)
                fbo = lax.bitcast_convert_type(vb & msk, jnp.float32)
                mrows[b, e, pl.ds(j * 32, 16)] = jnp.maximum(fae + fbe, 0.0)
                mrows[b, e, pl.ds(j * 32 + 16, 16)] = jnp.maximum(fao + fbo,
                                                                  0.0)

    def scatter_start(b, ib):
        pltpu.async_copy(mrows.at[b], aggsh.at[idxbuf.at[ib, 1]], ssc[b],
                         add=True)

    def scatter_wait(b):
        pltpu.make_async_copy(a_hbm.at[pl.ds(0, CH)], mrows.at[b],
                              ssc[b]).wait()

    def step(c, k, first):
        b = k % 2
        ib = k % 4
        ibn = (k + 2) % 4
        gather_wait(b)
        if not first:
            scatter_wait(b)
            cn2 = jnp.where(c + 2 >= ncw, 0, c + 2)
            idx_start(ibn, cn2)
        compute(b)
        scatter_start(b, ib)
        idx_wait(ibn)
        gather_start(b, ibn)

    for ib in range(4):
        idx_start(ib, jnp.int32(ib))
    idx_wait(0)
    gather_start(0, 0)
    idx_wait(1)
    gather_start(1, 1)
    step(jnp.int32(0), 0, True)
    step(jnp.int32(1), 1, True)
    step(jnp.int32(2), 2, False)
    step(jnp.int32(3), 3, False)

    def group(g, carry):
        for k in range(4):
            step(4 * g + k, k, False)
        return carry

    lax.fori_loop(1, ncw // 4, group, 0)
    for b in (0, 1):
        gather_wait(b)
        scatter_wait(b)
    plsc.subcore_barrier()

    off = 0
    for zn in ZSIZES:
        pltpu.sync_copy(aggsh.at[pl.ds(r0 + off, zn)],
                        out_hbm.at[cid, pl.ds(r0 + off, zn)])
        off += zn


@functools.lru_cache(maxsize=1)
def _sc_edge():
    return functools.partial(
        pl.kernel,
        mesh=plsc.VectorSubcoreMesh(core_axis_name="c", subcore_axis_name="s"),
        compiler_params=pltpu.CompilerParams(use_tc_tiling_on_sc=False),
        out_type=jax.ShapeDtypeStruct((NC, NPAD, H), jnp.float32),
        scratch_types=[
            pltpu.VMEM((4, 2, CH), jnp.int32),
            pltpu.VMEM((2, CH, H // 2), jnp.int32),
            pltpu.VMEM((2, CH, H // 2), jnp.int32),
            pltpu.VMEM((2, CH, H), jnp.float32),
            pltpu.VMEM_SHARED((NPAD, H), jnp.float32),
        ] + [pltpu.SemaphoreType.DMA] * 10,
    )(_sc_edge_body)


BLK = 1000
GRID = N // BLK


def _mm(a, b):
    return jnp.dot(a, b, preferred_element_type=jnp.float32)


def _enc_kernel(x_ref, w1, b1, w2, b2, wmt, wmb, bm, h_ref, a_ref, b_ref):
    h1 = jnp.maximum(_mm(x_ref[...], w1[...]) + b1[...], 0.0)
    h = jnp.maximum(_mm(h1, w2[...]) + b2[...], 0.0)
    h_ref[...] = h
    a_ref[...] = (_mm(h, wmt[...]) + bm[...]).astype(jnp.bfloat16)
    b_ref[...] = _mm(h, wmb[...]).astype(jnp.bfloat16)


def _upd_kernel(h_ref, agg_ref, wut, wub, bu, wmt, wmb, bm,
                hn_ref, a_ref, b_ref):
    agg = agg_ref[0] + agg_ref[1]
    hn = jnp.maximum(_mm(h_ref[...], wut[...]) + _mm(agg, wub[...]) + bu[...],
                     0.0)
    hn_ref[...] = hn
    a_ref[...] = (_mm(hn, wmt[...]) + bm[...]).astype(jnp.bfloat16)
    b_ref[...] = _mm(hn, wmb[...]).astype(jnp.bfloat16)


def _upd_last_kernel(h_ref, agg_ref, wut, wub, bu, wr1, br1, wr2, br2,
                     out_ref, gacc):
    i = pl.program_id(0)
    agg = agg_ref[0] + agg_ref[1]
    hn = jnp.maximum(_mm(h_ref[...], wut[...]) + _mm(agg, wub[...]) + bu[...],
                     0.0)
    part = jnp.sum(hn.reshape(BLK // 8, 8, H), axis=0)

    @pl.when(i == 0)
    def _init():
        gacc[...] = part

    @pl.when(i > 0)
    def _acc():
        gacc[...] = gacc[...] + part

    @pl.when(i == GRID - 1)
    def _readout():
        g = jnp.sum(gacc[...], axis=0, keepdims=True)
        lat = _mm(jnp.maximum(_mm(g, wr1[...]) + br1[...], 0.0), wr2[...])
        out_ref[...] = lat + br2[...]


def _row_spec():
    return pl.BlockSpec((BLK, H), lambda i: (i, 0))


def _full_spec(shape):
    return pl.BlockSpec(shape, lambda i: tuple(0 for _ in shape))


_NODE_SHAPE = jax.ShapeDtypeStruct((N, H), jnp.float32)
_NODE_SHAPE_BF = jax.ShapeDtypeStruct((N, H), jnp.bfloat16)

_enc_call = pl.pallas_call(
    _enc_kernel,
    grid=(GRID,),
    in_specs=[_row_spec()] + [_full_spec(s) for s in
                              [(D, H), (1, H), (H, H), (1, H),
                               (H, H), (H, H), (1, H)]],
    out_specs=[_row_spec(), _row_spec(), _row_spec()],
    out_shape=[_NODE_SHAPE, _NODE_SHAPE_BF, _NODE_SHAPE_BF],
)

_upd_call = pl.pallas_call(
    _upd_kernel,
    grid=(GRID,),
    in_specs=[_row_spec(),
              pl.BlockSpec((NC, BLK, H), lambda i: (0, i, 0))] +
             [_full_spec(s) for s in
              [(H, H), (H, H), (1, H), (H, H), (H, H), (1, H)]],
    out_specs=[_row_spec(), _row_spec(), _row_spec()],
    out_shape=[_NODE_SHAPE, _NODE_SHAPE_BF, _NODE_SHAPE_BF],
)

_upd_last_call = pl.pallas_call(
    _upd_last_kernel,
    grid=(GRID,),
    in_specs=[_row_spec(),
              pl.BlockSpec((NC, BLK, H), lambda i: (0, i, 0))] +
             [_full_spec(s) for s in
              [(H, H), (H, H), (1, H), (H, H), (1, H), (H, L), (1, L)]],
    out_specs=pl.BlockSpec((1, L), lambda i: (0, 0)),
    out_shape=jax.ShapeDtypeStruct((1, L), jnp.float32),
    scratch_shapes=[pltpu.VMEM((8, H), jnp.float32)],
)


def kernel(x, edge_index, W_enc1, b_enc1, W_enc2, b_enc2, W_msg, b_msg,
           W_upd, b_upd, W_r1, b_r1, W_r2, b_r2):
    src = edge_index[0].astype(jnp.int32)
    dst = edge_index[1].astype(jnp.int32)
    npd = EPAD - E
    dst_pad = N + jnp.arange(npd, dtype=jnp.int32) % (NPAD - N)
    src_p = jnp.concatenate([src, jnp.zeros((npd,), jnp.int32)])
    dst_p = jnp.concatenate([dst, dst_pad])
    idx_p = jnp.stack([src_p.reshape(NCHT, CH),
                       dst_p.reshape(NCHT, CH)], axis=1)

    wmt, wmb = W_msg[:H], W_msg[H:]
    wut, wub = W_upd[:H], W_upd[H:][PERM]
    b1 = b_enc1.reshape(1, H)
    b2 = b_enc2.reshape(1, H)
    bm = b_msg.reshape(1, H)
    bu = b_upd.reshape(1, H)
    br1 = b_r1.reshape(1, H)
    br2 = b_r2.reshape(1, L)

    h, a, b = _enc_call(x, W_enc1, b1, W_enc2, b2, wmt, wmb, bm)
    for it in range(ITERS):
        a32 = lax.bitcast_convert_type(a.reshape(N, H // 2, 2), jnp.int32)
        b32 = lax.bitcast_convert_type(b.reshape(N, H // 2, 2), jnp.int32)
        aggp = _sc_edge()(a32, b32, idx_p)
        if it < ITERS - 1:
            h, a, b = _upd_call(h, aggp, wut, wub, bu, wmt, wmb, bm)
        else:
            lat = _upd_last_call(h, aggp, wut, wub, bu,
                                 W_r1, br1, W_r2, br2)
    return lat.reshape(L)


# speedup vs baseline: 2.2195x; 1.2178x over previous
import functools

import jax
import jax.numpy as jnp
import numpy as np
from jax import lax
from jax.experimental import pallas as pl
from jax.experimental.pallas import tpu as pltpu
from jax.experimental.pallas import tpu_sc as plsc

N = 10000
E = 320000
D = 128
H = 128
L = 128
ITERS = 3

NC = 2
NS = 16
CH = 64
NW = NC * NS
CH_PER_W = 4 * (-(-E // (4 * CH * NW)))
EPAD = CH_PER_W * CH * NW
NCHT = EPAD // CH
CPW0 = 228
CPW1 = 2 * CH_PER_W - CPW0
NPAD = 10112

ROWS_PER_TILE = NPAD // NS
ZSIZES = [CH] * (ROWS_PER_TILE // CH) + (
    [ROWS_PER_TILE % CH] if ROWS_PER_TILE % CH else [])


def _sc_edge_body(a_hbm, b_hbm, idx_hbm, out_hbm,
                  idxbuf, arows, brows, mrows, aggsh,
                  sga0, sga1, sgb0, sgb1, ssc0, ssc1,
                  six0, six1, six2, six3):
    cid = lax.axis_index("c")
    sid = lax.axis_index("s")
    ncw = jnp.where(cid == 0, CPW0, CPW1)
    cbase = jnp.where(cid == 0, sid * CPW0, NS * CPW0 + sid * CPW1)
    sga = (sga0, sga1)
    sgb = (sgb0, sgb1)
    ssc = (ssc0, ssc1)
    six = (six0, six1, six2, six3)

    zero = jnp.zeros((16,), jnp.float32)

    def zrow(e, carry):
        for j in range(H // 16):
            mrows[0, e, pl.ds(j * 16, 16)] = zero
        return carry

    lax.fori_loop(0, CH, zrow, 0)
    r0 = sid * ROWS_PER_TILE
    off = 0
    for zn in ZSIZES:
        pltpu.sync_copy(mrows.at[0, pl.ds(0, zn)],
                        aggsh.at[pl.ds(r0 + off, zn)])
        off += zn
    plsc.subcore_barrier()

    def idx_start(ib, c):
        pltpu.async_copy(idx_hbm.at[cbase + c], idxbuf.at[ib], six[ib])

    def idx_wait(ib):
        pltpu.make_async_copy(idx_hbm.at[0], idxbuf.at[ib],
                              six[ib]).wait()

    def gather_start(b, ib):
        pltpu.async_copy(a_hbm.at[idxbuf.at[ib, 0]], arows.at[b], sga[b])
        pltpu.async_copy(b_hbm.at[idxbuf.at[ib, 1]], brows.at[b], sgb[b])

    def gather_wait(b):
        pltpu.make_async_copy(a_hbm.at[pl.ds(0, CH)], arows.at[b],
                              sga[b]).wait()
        pltpu.make_async_copy(b_hbm.at[pl.ds(0, CH)], brows.at[b],
                              sgb[b]).wait()

    def compute(b):
        msk = jnp.full((16,), -65536, jnp.int32)

        @plsc.parallel_loop(0, CH, 1, unroll=2)
        def edge(e):
            for j in range(H // 32):
                sw = pl.ds(j * 16, 16)
                va = arows[b, e, sw]
                vb = brows[b, e, sw]
                flo = lax.bitcast_convert_type(va << 16, jnp.float32)
                fhi = lax.bitcast_convert_type(va & msk, jnp.float32)
                glo = lax.bitcast_convert_type(vb << 16, jnp.float32)
                ghi = lax.bitcast_convert_type(vb & msk, jnp.float32)
                mrows[b, e, pl.ds(j * 16, 16)] = jnp.maximum(flo + glo, 0.0)
                mrows[b, e, pl.ds(H // 2 + j * 16, 16)] = jnp.maximum(
                    fhi + ghi, 0.0)

    def scatter_start(b, ib):
        pltpu.async_copy(mrows.at[b], aggsh.at[idxbuf.at[ib, 1]], ssc[b],
                         add=True)

    def scatter_wait(b):
        pltpu.make_async_copy(a_hbm.at[pl.ds(0, CH)], mrows.at[b],
                              ssc[b]).wait()

    def step(c, k, first):
        b = k % 2
        ib = k % 4
        ibn = (k + 2) % 4
        gather_wait(b)
        if not first:
            scatter_wait(b)
            cn2 = jnp.where(c + 2 >= ncw, 0, c + 2)
            idx_start(ibn, cn2)
        compute(b)
        scatter_start(b, ib)
        idx_wait(ibn)
        gather_start(b, ibn)

    for ib in range(4):
        idx_start(ib, jnp.int32(ib))
    idx_wait(0)
    gather_start(0, 0)
    idx_wait(1)
    gather_start(1, 1)
    step(jnp.int32(0), 0, True)
    step(jnp.int32(1), 1, True)
    step(jnp.int32(2), 2, False)
    step(jnp.int32(3), 3, False)

    def group(g, carry):
        for k in range(4):
            step(4 * g + k, k, False)
        return carry

    lax.fori_loop(1, ncw // 4, group, 0)
    for b in (0, 1):
        gather_wait(b)
        scatter_wait(b)
    plsc.subcore_barrier()

    off = 0
    for zn in ZSIZES:
        pltpu.sync_copy(aggsh.at[pl.ds(r0 + off, zn)],
                        out_hbm.at[cid, pl.ds(r0 + off, zn)])
        off += zn


@functools.lru_cache(maxsize=1)
def _sc_edge():
    return functools.partial(
        pl.kernel,
        mesh=plsc.VectorSubcoreMesh(core_axis_name="c", subcore_axis_name="s"),
        compiler_params=pltpu.CompilerParams(use_tc_tiling_on_sc=False),
        out_type=jax.ShapeDtypeStruct((NC, NPAD, H), jnp.float32),
        scratch_types=[
            pltpu.VMEM((4, 2, CH), jnp.int32),
            pltpu.VMEM((2, CH, H // 2), jnp.int32),
            pltpu.VMEM((2, CH, H // 2), jnp.int32),
            pltpu.VMEM((2, CH, H), jnp.float32),
            pltpu.VMEM_SHARED((NPAD, H), jnp.float32),
        ] + [pltpu.SemaphoreType.DMA] * 10,
    )(_sc_edge_body)


BLK = 1000
GRID = N // BLK


def _mm(a, b):
    return jnp.dot(a, b, preferred_element_type=jnp.float32)


def _pack_bf16(x):
    xf = x.astype(jnp.bfloat16).astype(jnp.float32)
    u = lax.bitcast_convert_type(xf, jnp.int32)
    lo = lax.shift_right_logical(u[:, :H // 2], 16)
    hi = u[:, H // 2:] & jnp.int32(-65536)
    return lo | hi


def _enc_kernel(x_ref, w1, b1, w2, b2, wmt, wmb, bm, h_ref, a_ref, b_ref):
    h1 = jnp.maximum(_mm(x_ref[...], w1[...]) + b1[...], 0.0)
    h = jnp.maximum(_mm(h1, w2[...]) + b2[...], 0.0)
    h_ref[...] = h
    a_ref[...] = _pack_bf16(_mm(h, wmt[...]) + bm[...])
    b_ref[...] = _pack_bf16(_mm(h, wmb[...]))


def _upd_kernel(h_ref, agg_ref, wut, wub, bu, wmt, wmb, bm,
                hn_ref, a_ref, b_ref):
    agg = agg_ref[0] + agg_ref[1]
    hn = jnp.maximum(_mm(h_ref[...], wut[...]) + _mm(agg, wub[...]) + bu[...],
                     0.0)
    hn_ref[...] = hn
    a_ref[...] = _pack_bf16(_mm(hn, wmt[...]) + bm[...])
    b_ref[...] = _pack_bf16(_mm(hn, wmb[...]))


def _upd_last_kernel(h_ref, agg_ref, wut, wub, bu, wr1, br1, wr2, br2,
                     out_ref, gacc):
    i = pl.program_id(0)
    agg = agg_ref[0] + agg_ref[1]
    hn = jnp.maximum(_mm(h_ref[...], wut[...]) + _mm(agg, wub[...]) + bu[...],
                     0.0)
    part = jnp.sum(hn.reshape(BLK // 8, 8, H), axis=0)

    @pl.when(i == 0)
    def _init():
        gacc[...] = part

    @pl.when(i > 0)
    def _acc():
        gacc[...] = gacc[...] + part

    @pl.when(i == GRID - 1)
    def _readout():
        g = jnp.sum(gacc[...], axis=0, keepdims=True)
        lat = _mm(jnp.maximum(_mm(g, wr1[...]) + br1[...], 0.0), wr2[...])
        out_ref[...] = lat + br2[...]


def _row_spec():
    return pl.BlockSpec((BLK, H), lambda i: (i, 0))


def _full_spec(shape):
    return pl.BlockSpec(shape, lambda i: tuple(0 for _ in shape))


_NODE_SHAPE = jax.ShapeDtypeStruct((N, H), jnp.float32)
_NODE_SHAPE_PK = jax.ShapeDtypeStruct((N, H // 2), jnp.int32)


def _pk_spec():
    return pl.BlockSpec((BLK, H // 2), lambda i: (i, 0))


_enc_call = pl.pallas_call(
    _enc_kernel,
    grid=(GRID,),
    in_specs=[_row_spec()] + [_full_spec(s) for s in
                              [(D, H), (1, H), (H, H), (1, H),
                               (H, H), (H, H), (1, H)]],
    out_specs=[_row_spec(), _pk_spec(), _pk_spec()],
    out_shape=[_NODE_SHAPE, _NODE_SHAPE_PK, _NODE_SHAPE_PK],
)

_upd_call = pl.pallas_call(
    _upd_kernel,
    grid=(GRID,),
    in_specs=[_row_spec(),
              pl.BlockSpec((NC, BLK, H), lambda i: (0, i, 0))] +
             [_full_spec(s) for s in
              [(H, H), (H, H), (1, H), (H, H), (H, H), (1, H)]],
    out_specs=[_row_spec(), _pk_spec(), _pk_spec()],
    out_shape=[_NODE_SHAPE, _NODE_SHAPE_PK, _NODE_SHAPE_PK],
)

_upd_last_call = pl.pallas_call(
    _upd_last_kernel,
    grid=(GRID,),
    in_specs=[_row_spec(),
              pl.BlockSpec((NC, BLK, H), lambda i: (0, i, 0))] +
             [_full_spec(s) for s in
              [(H, H), (H, H), (1, H), (H, H), (1, H), (H, L), (1, L)]],
    out_specs=pl.BlockSpec((1, L), lambda i: (0, 0)),
    out_shape=jax.ShapeDtypeStruct((1, L), jnp.float32),
    scratch_shapes=[pltpu.VMEM((8, H), jnp.float32)],
)


def kernel(x, edge_index, W_enc1, b_enc1, W_enc2, b_enc2, W_msg, b_msg,
           W_upd, b_upd, W_r1, b_r1, W_r2, b_r2):
    src = edge_index[0].astype(jnp.int32)
    dst = edge_index[1].astype(jnp.int32)
    npd = EPAD - E
    dst_pad = N + jnp.arange(npd, dtype=jnp.int32) % (NPAD - N)
    src_p = jnp.concatenate([src, jnp.zeros((npd,), jnp.int32)])
    dst_p = jnp.concatenate([dst, dst_pad])
    idx_p = jnp.stack([src_p.reshape(NCHT, CH),
                       dst_p.reshape(NCHT, CH)], axis=1)

    wmt, wmb = W_msg[:H], W_msg[H:]
    wut, wub = W_upd[:H], W_upd[H:]
    b1 = b_enc1.reshape(1, H)
    b2 = b_enc2.reshape(1, H)
    bm = b_msg.reshape(1, H)
    bu = b_upd.reshape(1, H)
    br1 = b_r1.reshape(1, H)
    br2 = b_r2.reshape(1, L)

    h, a, b = _enc_call(x, W_enc1, b1, W_enc2, b2, wmt, wmb, bm)
    for it in range(ITERS):
        aggp = _sc_edge()(a, b, idx_p)
        if it < ITERS - 1:
            h, a, b = _upd_call(h, aggp, wut, wub, bu, wmt, wmb, bm)
        else:
            lat = _upd_last_call(h, aggp, wut, wub, bu,
                                 W_r1, br1, W_r2, br2)
    return lat.reshape(L)
